# baseline (speedup 1.0000x reference)
import functools

import jax
import jax.numpy as jnp
from jax import lax
from jax.experimental import pallas as pl
from jax.experimental.pallas import tpu as pltpu
from jax.experimental.pallas import tpu_sc as plsc

NC = 2
NS = 16
NW = NC * NS

NG = 3
L = 16


def _body(uids_t_hbm, keys_hbm, out5_hbm, idx_v, rows_v, stg_v, gsem, osem):
    wid = lax.axis_index("c") * NS + lax.axis_index("s")
    hist = idx_v.shape[0]
    key_dim = rows_v.shape[2]
    iota = lax.iota(jnp.int32, L)

    pltpu.sync_copy(uids_t_hbm.at[:, pl.ds(wid * 128, 128)], idx_v)

    def fire(h):
        pltpu.async_copy(
            keys_hbm.at[idx_v.at[h]], rows_v.at[lax.rem(h, NG)], gsem
        )

    def wait_gather():
        pltpu.make_async_copy(
            keys_hbm.at[pl.ds(0, 128)], rows_v.at[0], gsem
        ).wait()

    def wait_outs():
        for _ in range(8):
            pltpu.make_async_copy(
                stg_v.at[0, 0], out5_hbm.at[0, 0, 0], osem
            ).wait()

    for h in range(NG):
        fire(h)

    def step(h, carry):
        wait_gather()

        @pl.when(h >= 2)
        def _():
            wait_outs()

        gbuf = lax.rem(h, NG)
        sbuf = lax.rem(h, 2)
        gsplat = jnp.full((L,), gbuf, jnp.int32)

        def transpose_d(d, c2):
            dsplat = jnp.full((L,), d, jnp.int32)
            dr = d // 8
            s = lax.rem(d, 8)
            for c in range(8):
                vec = plsc.load_gather(
                    rows_v, [gsplat, c * L + iota, dsplat]
                )
                stg_v[sbuf, dr, s, pl.ds(c * L, L)] = vec
            return c2

        lax.fori_loop(0, key_dim, transpose_d, 0, unroll=8)

        for dr in range(8):
            pltpu.async_copy(
                stg_v.at[sbuf, dr], out5_hbm.at[h, dr, wid], osem
            )

        @pl.when(h + NG < hist)
        def _():
            fire(h + NG)

        return carry

    lax.fori_loop(0, hist, step, 0, unroll=False)
    wait_outs()
    wait_outs()


def kernel(uids, keys):
    batch, hist = uids.shape
    key_dim = keys.shape[1]
    assert batch == NW * 128 and key_dim == 64

    mesh = plsc.VectorSubcoreMesh(core_axis_name="c", subcore_axis_name="s")
    out5 = pl.kernel(
        _body,
        out_type=jax.ShapeDtypeStruct((hist, 8, NW, 8, 128), keys.dtype),
        mesh=mesh,
        scratch_types=[
            pltpu.VMEM((hist, 128), jnp.int32),
            pltpu.VMEM((NG, 128, key_dim), keys.dtype),
            pltpu.VMEM((2, 8, 8, 128), keys.dtype),
            pltpu.SemaphoreType.DMA,
            pltpu.SemaphoreType.DMA,
        ],
        compiler_params=pltpu.CompilerParams(
            use_tc_tiling_on_sc=False, needs_layout_passes=False
        ),
    )(uids.T, keys)
    return out5.transpose(2, 4, 0, 1, 3).reshape(batch, hist, key_dim)


# speedup vs baseline: 1.2176x; 1.2176x over previous
import functools

import jax
import jax.numpy as jnp
from jax import lax
from jax.experimental import pallas as pl
from jax.experimental.pallas import tpu as pltpu
from jax.experimental.pallas import tpu_sc as plsc

NC = 2
NS = 16
NW = NC * NS

NG = 4


def _body(uids_t_hbm, keys_hbm, out_t_hbm, idx_v, rows_v, gsem, osem):
    wid = lax.axis_index("c") * NS + lax.axis_index("s")
    hist = idx_v.shape[0]
    base = wid * 128

    pltpu.sync_copy(uids_t_hbm.at[:, pl.ds(base, 128)], idx_v)

    def fire(h):
        pltpu.async_copy(
            keys_hbm.at[idx_v.at[h]], rows_v.at[lax.rem(h, NG)], gsem
        )

    def wait_gather():
        pltpu.make_async_copy(
            keys_hbm.at[pl.ds(0, 128)], rows_v.at[0], gsem
        ).wait()

    def wait_out():
        pltpu.make_async_copy(
            rows_v.at[0], out_t_hbm.at[0, pl.ds(0, 128)], osem
        ).wait()

    for h in range(NG - 1):
        fire(h)

    def step(h, carry):
        wait_gather()
        pltpu.async_copy(
            rows_v.at[lax.rem(h, NG)], out_t_hbm.at[h, pl.ds(base, 128)], osem
        )

        @pl.when(h >= 1)
        def _():
            wait_out()

        @pl.when(h + NG - 1 < hist)
        def _():
            fire(h + NG - 1)

        return carry

    lax.fori_loop(0, hist, step, 0, unroll=False)
    wait_out()


def kernel(uids, keys):
    batch, hist = uids.shape
    key_dim = keys.shape[1]
    assert batch == NW * 128

    mesh = plsc.VectorSubcoreMesh(core_axis_name="c", subcore_axis_name="s")
    out_t = pl.kernel(
        _body,
        out_type=jax.ShapeDtypeStruct((hist, batch, key_dim), keys.dtype),
        mesh=mesh,
        scratch_types=[
            pltpu.VMEM((hist, 128), jnp.int32),
            pltpu.VMEM((NG, 128, key_dim), keys.dtype),
            pltpu.SemaphoreType.DMA,
            pltpu.SemaphoreType.DMA,
        ],
        compiler_params=pltpu.CompilerParams(use_tc_tiling_on_sc=False),
    )(uids.T, keys)
    return out_t.transpose(1, 0, 2)
